# baseline (speedup 1.0000x reference)
import functools

import jax
import jax.numpy as jnp
from jax import lax
from jax.experimental import pallas as pl
from jax.experimental.pallas import tpu as pltpu
from jax.experimental.pallas import tpu_sc as plsc

_BATCH = 16384
_HIST = 50
_D = 32

_NC = 2
_NS = 16
_NW = _NC * _NS
_BPW = _BATCH // _NW
_NBUF = 5
_N_OUTER = _HIST // _NBUF

_mesh = plsc.VectorSubcoreMesh(core_axis_name="c", subcore_axis_name="s")


@functools.partial(
    pl.kernel,
    out_type=jax.ShapeDtypeStruct((_HIST, _BATCH, _D), jnp.float32),
    mesh=_mesh,
    scratch_types=[
        pltpu.VMEM((_HIST, _BPW), jnp.int32),
        pltpu.VMEM((_NBUF, _BPW, _D), jnp.float32),
        pltpu.SemaphoreType.DMA((_NBUF,)),
        pltpu.SemaphoreType.DMA((_NBUF,)),
    ],
    compiler_params=pltpu.CompilerParams(
        use_tc_tiling_on_sc=False, needs_layout_passes=False
    ),
)
def _sc_gather(idxt_hbm, table_hbm, out_hbm, idx_v, rows_v, gsem, ssem):
    wid = lax.axis_index("s") * _NC + lax.axis_index("c")
    b0 = wid * _BPW
    pltpu.sync_copy(idxt_hbm.at[:, pl.ds(b0, _BPW)], idx_v)

    def start_gather(h, b):
        for c in range(4):
            pltpu.make_async_copy(
                table_hbm.at[idx_v.at[h, pl.ds(c * 128, 128)]],
                rows_v.at[b, pl.ds(c * 128, 128)],
                gsem.at[b],
            ).start()

    def wait_gather(h, b):
        pltpu.make_async_copy(
            out_hbm.at[h, pl.ds(b0, _BPW)], rows_v.at[b], gsem.at[b]
        ).wait()

    def store_desc(h, b):
        return pltpu.make_async_copy(
            rows_v.at[b],
            out_hbm.at[h, pl.ds(b0, _BPW)],
            ssem.at[b],
        )

    for b in range(_NBUF - 1):
        start_gather(b, b)

    def outer(i, carry):
        t0 = i * _NBUF
        for b in range(_NBUF):
            t = t0 + b
            h_next = t + _NBUF - 1
            bn = (b - 1) % _NBUF

            def refill(t=t, h_next=h_next, bn=bn, guard_prev=(b == 0)):
                if guard_prev:
                    @pl.when(t >= 1)
                    def _():
                        store_desc(t - 1, bn).wait()
                else:
                    store_desc(t - 1, bn).wait()
                start_gather(h_next, bn)

            pl.when(h_next < _HIST)(refill)

            wait_gather(t, b)
            store_desc(t, b).start()
        return carry

    lax.fori_loop(0, _N_OUTER, outer, 0)

    for b in range(_NBUF):
        store_desc(_HIST - _NBUF + b, b).wait()


def kernel(inputs, table):
    return _sc_gather(inputs.T, table).transpose(1, 0, 2)


# speedup vs baseline: 1.0030x; 1.0030x over previous
import functools

import jax
import jax.numpy as jnp
from jax import lax
from jax.experimental import pallas as pl
from jax.experimental.pallas import tpu as pltpu
from jax.experimental.pallas import tpu_sc as plsc

_BATCH = 16384
_HIST = 50
_D = 32

_NC = 2
_NS = 16
_NW = _NC * _NS
_BPW = _BATCH // _NW
_NBUF = 5
_N_OUTER = _HIST // _NBUF

_mesh = plsc.VectorSubcoreMesh(core_axis_name="c", subcore_axis_name="s")


@functools.partial(
    pl.kernel,
    out_type=jax.ShapeDtypeStruct((_HIST, _BATCH, _D), jnp.float32),
    mesh=_mesh,
    scratch_types=[
        pltpu.VMEM((_HIST, _BPW), jnp.int32),
        pltpu.VMEM((_NBUF, _BPW, _D), jnp.float32),
        pltpu.SemaphoreType.DMA((_NBUF,)),
        pltpu.SemaphoreType.DMA((_NBUF,)),
    ],
    compiler_params=pltpu.CompilerParams(
        use_tc_tiling_on_sc=False, needs_layout_passes=False
    ),
)
def _sc_gather(idxt_hbm, table_hbm, out_hbm, idx_v, rows_v, gsem, ssem):
    wid = lax.axis_index("s") * _NC + lax.axis_index("c")
    b0 = wid * _BPW
    pltpu.sync_copy(idxt_hbm.at[:, pl.ds(b0, _BPW)], idx_v)

    def start_gather(h, b):
        for c in range(4):
            pltpu.make_async_copy(
                table_hbm.at[idx_v.at[h, pl.ds(c * 128, 128)]],
                rows_v.at[b, pl.ds(c * 128, 128)],
                gsem.at[b],
            ).start()

    def wait_gather(h, b):
        pltpu.make_async_copy(
            out_hbm.at[h, pl.ds(b0, _BPW)], rows_v.at[b], gsem.at[b]
        ).wait()

    def store_desc(h, b):
        return pltpu.make_async_copy(
            rows_v.at[b],
            out_hbm.at[h, pl.ds(b0, _BPW)],
            ssem.at[b],
        )

    for b in range(_NBUF - 1):
        start_gather(b, b)

    def outer(i, carry):
        t0 = i * _NBUF
        for b in range(_NBUF):
            t = t0 + b
            h_next = t + _NBUF - 1
            bn = (b - 1) % _NBUF

            def refill(t=t, h_next=h_next, bn=bn, guard_prev=(b == 0)):
                if guard_prev:
                    @pl.when(t >= 1)
                    def _():
                        store_desc(t - 1, bn).wait()
                else:
                    store_desc(t - 1, bn).wait()
                start_gather(h_next, bn)

            pl.when(h_next < _HIST)(refill)

            wait_gather(t, b)
            store_desc(t, b).start()
        return carry

    lax.fori_loop(0, _N_OUTER, outer, 0)

    for b in range(_NBUF):
        store_desc(_HIST - _NBUF + b, b).wait()


def kernel(inputs, table):
    idx_t = inputs.astype(jnp.int32).T
    return _sc_gather(idx_t, table).transpose(1, 0, 2)
